# baseline (speedup 1.0000x reference)
import functools

import jax
import jax.numpy as jnp
from jax import lax
from jax.experimental import pallas as pl
from jax.experimental.pallas import tpu as pltpu
from jax.experimental.pallas import tpu_sc as plsc

_CHUNK = 128


@functools.partial(jax.jit, static_argnums=(2, 3, 4))
def _gather(idx, table, B, D, NW):
    b_per_w = B // NW
    n_chunks = b_per_w // _CHUNK
    idx3 = idx.reshape(NW, n_chunks, _CHUNK)
    mesh = plsc.VectorSubcoreMesh(core_axis_name="c", subcore_axis_name="s")

    @functools.partial(
        pl.kernel,
        mesh=mesh,
        out_type=jax.ShapeDtypeStruct((B, D), jnp.float32),
        scratch_types=[
            pltpu.VMEM((n_chunks, _CHUNK), jnp.int32),
            pltpu.VMEM((b_per_w, D), jnp.float32),
            pltpu.SemaphoreType.DMA,
        ],
        compiler_params=pltpu.CompilerParams(use_tc_tiling_on_sc=False),
    )
    def k(idx_hbm, table_hbm, out_hbm, idx_v, rows_v, sem):
        wid = lax.axis_index("s") * 2 + lax.axis_index("c")
        base = wid * b_per_w
        pltpu.sync_copy(idx_hbm.at[wid], idx_v)
        copies = [
            pltpu.async_copy(
                table_hbm.at[idx_v.at[j]],
                rows_v.at[pl.ds(j * _CHUNK, _CHUNK)],
                sem,
            )
            for j in range(n_chunks)
        ]
        for c in copies:
            c.wait()
        pltpu.sync_copy(rows_v, out_hbm.at[pl.ds(base, b_per_w)])

    return k(idx3, table)


def kernel(user_indices, visual_tokens):
    B = user_indices.shape[0]
    V, T, D = visual_tokens.shape
    table = visual_tokens.reshape(V * T, D)
    idx = user_indices.astype(jnp.int32)
    out = _gather(idx, table, B, D, 32)
    return out.reshape(B, T, D)


# speedup vs baseline: 1.5263x; 1.5263x over previous
import functools

import jax
import jax.numpy as jnp
from jax import lax
from jax.experimental import pallas as pl
from jax.experimental.pallas import tpu as pltpu
from jax.experimental.pallas import tpu_sc as plsc

_B = 16384
_D = 64
_V = 1000000
_NSLAB = _V // 128 + 1
_TAIL_C = _NSLAB - 1
_TAIL_U = _TAIL_C * 128
_PAIRS = 123


def _iota16():
    return lax.iota(jnp.int32, 16)


@jax.jit
def _gather_scan(idx, table_t, tail):
    mesh = plsc.VectorSubcoreMesh(core_axis_name="c", subcore_axis_name="s")

    @functools.partial(
        pl.kernel,
        mesh=mesh,
        out_type=jax.ShapeDtypeStruct((_B + 128, 128), jnp.float32),
        scratch_types=[
            pltpu.VMEM((_B,), jnp.int32),
            pltpu.VMEM((_B + 16,), jnp.int32),
            pltpu.VMEM((_B + 16,), jnp.int32),
            pltpu.VMEM((_D, 128), jnp.float32),
            pltpu.VMEM((_D, 128), jnp.float32),
            pltpu.VMEM((128, 128), jnp.float32),
            pltpu.VMEM((128,), jnp.int32),
            pltpu.SemaphoreType.DMA,
            pltpu.SemaphoreType.DMA,
            pltpu.SemaphoreType.DMA,
        ],
        compiler_params=pltpu.CompilerParams(needs_layout_passes=False),
    )
    def k(idx_hbm, table_hbm, tail_hbm, out_hbm,
          idx_v, mu_v, mb_v, slab0, slab1, stage_v, sl_v,
          sem0, sem1, sem_s):
        wid = lax.axis_index("s") * 2 + lax.axis_index("c")
        c0 = 244 * wid + jnp.minimum(wid, 5)
        cn = jnp.where(wid < 5, 245, 244)
        cend = c0 + cn

        def init_sl(i, _):
            sl_v[pl.ds(i * 16, 16)] = jnp.full((16,), _B, jnp.int32)
            return 0
        lax.fori_loop(0, 8, init_sl, 0)

        pltpu.sync_copy(idx_hbm, idx_v)

        def scan_body(i, cnt):
            u = idx_v[pl.ds(i * 16, 16)]
            cc = lax.shift_right_logical(u, 7)
            mine = (cc >= c0) & (cc < cend)
            pos = cnt + plsc.cumsum(mine.astype(jnp.int32)) - 1
            plsc.store_scatter(mu_v, [pos], u, mask=mine)
            plsc.store_scatter(mb_v, [pos], _iota16() + i * 16, mask=mine)
            return cnt + jnp.max(plsc.all_reduce_population_count(mine))

        m = lax.fori_loop(0, _B // 16, scan_body, 0)
        qmax = lax.shift_right_logical(m + 15, 4)

        def fire(jj, slot_ref, sem):
            c = c0 + jj

            @pl.when(c == _TAIL_C)
            def _():
                pltpu.async_copy(tail_hbm, slot_ref, sem)

            @pl.when(c != _TAIL_C)
            def _():
                pltpu.async_copy(
                    table_hbm.at[:, pl.ds(pl.multiple_of(c * 128, 128), 128)],
                    slot_ref, sem)

        def flush_reset(_sp):
            pltpu.async_copy(stage_v, out_hbm.at[sl_v], sem_s).wait()
            return 0

        def process_slab(jj, slot_ref, sp):
            c = c0 + jj

            def qbody(q, sp):
                valid = (_iota16() + q * 16) < m
                uu = mu_v[pl.ds(q * 16, 16)]
                sel = valid & (lax.shift_right_logical(uu, 7) == c)
                n = jnp.max(plsc.all_reduce_population_count(sel))

                def do_group(sp):
                    bb = mb_v[pl.ds(q * 16, 16)]
                    colv = uu & 127
                    cs = plsc.cumsum(sel.astype(jnp.int32))
                    slot = sp + cs - 1
                    for f in range(_D):
                        fvec = jnp.full((16,), f, jnp.int32)
                        vals = plsc.load_gather(
                            slot_ref, [fvec, colv], mask=sel)
                        plsc.store_scatter(
                            stage_v, [slot, fvec], vals, mask=sel)
                    plsc.store_scatter(sl_v, [slot], bb, mask=sel)
                    sp = sp + n
                    return lax.cond(sp >= 112, flush_reset,
                                    lambda s: s, sp)

                return lax.cond(n > 0, do_group, lambda s: s, sp)

            return lax.fori_loop(0, qmax, qbody, sp)

        fire(0, slab0, sem0)
        fire(1, slab1, sem1)

        def make_step(slot_ref, sem):
            def step(jj, sp):
                pltpu.make_async_copy(
                    table_hbm.at[:, pl.ds(0, 128)], slot_ref, sem).wait()
                sp = process_slab(jj, slot_ref, sp)

                @pl.when(jj + 2 < cn)
                def _():
                    fire(jj + 2, slot_ref, sem)
                return sp
            return step

        step0 = make_step(slab0, sem0)
        step1 = make_step(slab1, sem1)

        def pair_body(p, sp):
            sp = lax.cond(2 * p < cn,
                          lambda s: step0(2 * p, s), lambda s: s, sp)
            sp = lax.cond(2 * p + 1 < cn,
                          lambda s: step1(2 * p + 1, s), lambda s: s, sp)
            return sp

        sp = lax.fori_loop(0, _PAIRS, pair_body, 0)
        pltpu.async_copy(stage_v, out_hbm.at[sl_v], sem_s).wait()

    return k(idx, table_t, tail)


def kernel(user_indices, visual_tokens):
    B = user_indices.shape[0]
    V, T, D = visual_tokens.shape
    table_t = visual_tokens.reshape(V * T, D).T
    tail = jnp.pad(table_t[:, _TAIL_U:], ((0, 0), (0, 128 - (V - _TAIL_U))))
    idx = user_indices.astype(jnp.int32)
    out3 = _gather_scan(idx, table_t, tail)
    return out3[:B, :D].reshape(B, T, D)


# speedup vs baseline: 2.0538x; 1.3456x over previous
import functools

import jax
import jax.numpy as jnp
from jax import lax
from jax.experimental import pallas as pl
from jax.experimental.pallas import tpu as pltpu
from jax.experimental.pallas import tpu_sc as plsc

_B = 16384
_D = 64
_V = 1000000
_NSLAB = _V // 128 + 1
_TAIL_C = _NSLAB - 1
_TAIL_U = _TAIL_C * 128
_GP = 62


def _iota16():
    return lax.iota(jnp.int32, 16)


@jax.jit
def _gather_scan(idx, table_t, tail):
    mesh = plsc.VectorSubcoreMesh(core_axis_name="c", subcore_axis_name="s")

    @functools.partial(
        pl.kernel,
        mesh=mesh,
        out_type=jax.ShapeDtypeStruct((_B + 128, 128), jnp.float32),
        scratch_types=[
            pltpu.VMEM((_B,), jnp.int32),
            pltpu.VMEM((_B + 16,), jnp.int32),
            pltpu.VMEM((_B + 16,), jnp.int32),
            pltpu.VMEM((4096,), jnp.int32),
            pltpu.VMEM((256,), jnp.int32),
            pltpu.VMEM((_D, 256), jnp.float32),
            pltpu.VMEM((_D, 256), jnp.float32),
            pltpu.VMEM((128, 128), jnp.float32),
            pltpu.VMEM((128,), jnp.int32),
            pltpu.SemaphoreType.DMA,
            pltpu.SemaphoreType.DMA,
            pltpu.SemaphoreType.DMA,
        ],
        compiler_params=pltpu.CompilerParams(needs_layout_passes=False),
    )
    def k(idx_hbm, table_hbm, tail_hbm, out_hbm,
          idx_v, mu_v, mb_v, bucket_v, fill_v, buf0, buf1, stage_v, sl_v,
          sem0, sem1, sem_s):
        wid = lax.axis_index("s") * 2 + lax.axis_index("c")
        c0 = 244 * wid + jnp.minimum(wid, 5)
        cn = jnp.where(wid < 5, 245, 244)
        cend = c0 + cn
        ngroups = lax.shift_right_logical(cn + 1, 1)

        def fire_group(g, buf, sem):
            c = c0 + 2 * g

            @pl.when(c + 1 == _TAIL_C)
            def _():
                pltpu.async_copy(
                    table_hbm.at[:, pl.ds(pl.multiple_of(c * 128, 128), 128)],
                    buf.at[:, pl.ds(0, 128)], sem)
                pltpu.async_copy(tail_hbm, buf.at[:, pl.ds(128, 128)], sem)

            @pl.when(c + 1 != _TAIL_C)
            def _():
                pltpu.async_copy(
                    table_hbm.at[:, pl.ds(pl.multiple_of(c * 128, 128), 256)],
                    buf, sem)

        fire_group(0, buf0, sem0)
        fire_group(1, buf1, sem1)

        def init_sl(i, _):
            sl_v[pl.ds(i * 16, 16)] = jnp.full((16,), _B, jnp.int32)
            return 0
        lax.fori_loop(0, 8, init_sl, 0)

        def init_fill(i, _):
            fill_v[pl.ds(i * 16, 16)] = jnp.zeros((16,), jnp.int32)
            return 0
        lax.fori_loop(0, 16, init_fill, 0)

        pltpu.sync_copy(idx_hbm, idx_v)

        def bin_body(i, _):
            u = idx_v[pl.ds(i * 16, 16)]
            cc = lax.shift_right_logical(u, 7)
            mine = (cc >= c0) & (cc < cend)
            lb = cc - c0
            rank, last = plsc.scan_count(lb, mask=mine)
            base = plsc.load_gather(fill_v, [lb], mask=mine)
            slotb = base + rank - 1
            inb = mine & (slotb < 16)
            pack = (_iota16() + i * 16) | ((u & 127) << 14)
            plsc.store_scatter(bucket_v, [lb * 16 + slotb], pack, mask=inb)
            plsc.store_scatter(fill_v, [lb], base + rank, mask=mine & last)
            return 0

        lax.fori_loop(0, _B // 16, bin_body, 0)

        def ov_body(q, s):
            f16 = fill_v[pl.ds(q * 16, 16)]
            return s + jnp.sum(jnp.maximum(f16 - 16, 0))

        m_ov = lax.fori_loop(0, 16, ov_body, 0)

        def flush_reset(_sp):
            pltpu.async_copy(stage_v, out_hbm.at[sl_v], sem_s).wait()
            return 0

        def bucket_proc(lb, h, buf, sp):
            nh = plsc.load_gather(fill_v, [jnp.full((16,), 0, jnp.int32) + lb])
            sel = _iota16() < nh
            n = jnp.max(jnp.minimum(nh, 16))

            def do_group(sp):
                pp = bucket_v[pl.ds(lb * 16, 16)]
                bb = pp & 16383
                colv = lax.shift_right_logical(pp, 14) + h * 128
                slot = sp + plsc.cumsum(sel.astype(jnp.int32)) - 1
                for f in range(_D):
                    fvec = jnp.full((16,), f, jnp.int32)
                    vals = plsc.load_gather(buf, [fvec, colv], mask=sel)
                    plsc.store_scatter(stage_v, [slot, fvec], vals, mask=sel)
                plsc.store_scatter(sl_v, [slot], bb, mask=sel)
                sp = sp + n
                return lax.cond(sp >= 112, flush_reset, lambda s: s, sp)

            return lax.cond(n > 0, do_group, lambda s: s, sp)

        def process_group(g, buf, sp):
            sp = bucket_proc(2 * g, 0, buf, sp)
            lb1 = 2 * g + 1
            return lax.cond(lb1 < cn,
                            lambda s: bucket_proc(lb1, 1, buf, s),
                            lambda s: s, sp)

        def make_step(buf, sem):
            def step(g, sp):
                pltpu.make_async_copy(
                    table_hbm.at[:, pl.ds(0, 256)], buf, sem).wait()
                sp = process_group(g, buf, sp)

                @pl.when(g + 2 < ngroups)
                def _():
                    fire_group(g + 2, buf, sem)
                return sp
            return step

        step0 = make_step(buf0, sem0)
        step1 = make_step(buf1, sem1)

        def pair_body(p, sp):
            sp = lax.cond(2 * p < ngroups,
                          lambda s: step0(2 * p, s), lambda s: s, sp)
            sp = lax.cond(2 * p + 1 < ngroups,
                          lambda s: step1(2 * p + 1, s), lambda s: s, sp)
            return sp

        sp = lax.fori_loop(0, _GP, pair_body, 0)

        def fallback(sp):
            def scan_body(i, cnt):
                u = idx_v[pl.ds(i * 16, 16)]
                cc = lax.shift_right_logical(u, 7)
                mine = (cc >= c0) & (cc < cend)
                pos = cnt + plsc.cumsum(mine.astype(jnp.int32)) - 1
                plsc.store_scatter(mu_v, [pos], u, mask=mine)
                plsc.store_scatter(mb_v, [pos], _iota16() + i * 16,
                                   mask=mine)
                return cnt + jnp.max(plsc.all_reduce_population_count(mine))

            m = lax.fori_loop(0, _B // 16, scan_body, 0)
            qmax = lax.shift_right_logical(m + 15, 4)

            def slab_body(jj, sp):
                c = c0 + jj

                @pl.when(c == _TAIL_C)
                def _():
                    pltpu.sync_copy(tail_hbm, buf0.at[:, pl.ds(0, 128)])

                @pl.when(c != _TAIL_C)
                def _():
                    pltpu.sync_copy(
                        table_hbm.at[
                            :, pl.ds(pl.multiple_of(c * 128, 128), 128)],
                        buf0.at[:, pl.ds(0, 128)])

                def qbody(q, sp):
                    valid = (_iota16() + q * 16) < m
                    uu = mu_v[pl.ds(q * 16, 16)]
                    sel = valid & (lax.shift_right_logical(uu, 7) == c)
                    n = jnp.max(plsc.all_reduce_population_count(sel))

                    def do_group(sp):
                        bb = mb_v[pl.ds(q * 16, 16)]
                        colv = uu & 127
                        slot = sp + plsc.cumsum(sel.astype(jnp.int32)) - 1
                        for f in range(_D):
                            fvec = jnp.full((16,), f, jnp.int32)
                            vals = plsc.load_gather(
                                buf0, [fvec, colv], mask=sel)
                            plsc.store_scatter(
                                stage_v, [slot, fvec], vals, mask=sel)
                        plsc.store_scatter(sl_v, [slot], bb, mask=sel)
                        sp = sp + n
                        return lax.cond(sp >= 112, flush_reset,
                                        lambda s: s, sp)

                    return lax.cond(n > 0, do_group, lambda s: s, sp)

                return lax.fori_loop(0, qmax, qbody, sp)

            return lax.fori_loop(0, cn, slab_body, sp)

        sp = lax.cond(m_ov > 0, fallback, lambda s: s, sp)

        pltpu.async_copy(stage_v, out_hbm.at[sl_v], sem_s).wait()

    return k(idx, table_t, tail)


def kernel(user_indices, visual_tokens):
    B = user_indices.shape[0]
    V, T, D = visual_tokens.shape
    table_t = visual_tokens.reshape(V * T, D).T
    tail = jnp.pad(table_t[:, _TAIL_U:], ((0, 0), (0, 128 - (V - _TAIL_U))))
    idx = user_indices.astype(jnp.int32)
    out3 = _gather_scan(idx, table_t, tail)
    return out3[:B, :D].reshape(B, T, D)


# speedup vs baseline: 3.3033x; 1.6084x over previous
import functools

import jax
import jax.numpy as jnp
from jax import lax
from jax.experimental import pallas as pl
from jax.experimental.pallas import tpu as pltpu
from jax.experimental.pallas import tpu_sc as plsc

_B = 16384
_D = 64
_V = 1000000
_NSLAB = _V // 128 + 1
_TAIL_C = _NSLAB - 1
_TAIL_U = _TAIL_C * 128
_GP = 31


def _iota16():
    return lax.iota(jnp.int32, 16)


@jax.jit
def _gather_scan(idx, table_t, tail):
    mesh = plsc.VectorSubcoreMesh(core_axis_name="c", subcore_axis_name="s")

    @functools.partial(
        pl.kernel,
        mesh=mesh,
        out_type=jax.ShapeDtypeStruct((_B + 128, 128), jnp.float32),
        scratch_types=[
            pltpu.VMEM((_B,), jnp.int32),
            pltpu.VMEM((4096,), jnp.int32),
            pltpu.VMEM((256,), jnp.int32),
            pltpu.VMEM((_D, 512), jnp.float32),
            pltpu.VMEM((_D, 512), jnp.float32),
            pltpu.VMEM((128, 128), jnp.float32),
            pltpu.VMEM((128,), jnp.int32),
            pltpu.SemaphoreType.DMA,
            pltpu.SemaphoreType.DMA,
            pltpu.SemaphoreType.DMA,
        ],
        compiler_params=pltpu.CompilerParams(needs_layout_passes=False),
    )
    def k(idx_hbm, table_hbm, tail_hbm, out_hbm,
          idx_v, bucket_v, fill_v, buf0, buf1, stage_v, sl_v,
          sem0, sem1, sem_s):
        wid = lax.axis_index("s") * 2 + lax.axis_index("c")
        c0 = 244 * wid + jnp.minimum(wid, 5)
        cn = jnp.where(wid < 5, 245, 244)
        cend = c0 + cn
        ngroups = lax.shift_right_logical(cn + 3, 2)

        def fire_group(g, buf, sem):
            c = c0 + 4 * g

            @pl.when(c + 3 == _TAIL_C)
            def _():
                pltpu.async_copy(
                    table_hbm.at[:, pl.ds(pl.multiple_of(c * 128, 128), 384)],
                    buf.at[:, pl.ds(0, 384)], sem)
                pltpu.async_copy(tail_hbm, buf.at[:, pl.ds(384, 128)], sem)

            @pl.when(c + 3 != _TAIL_C)
            def _():
                pltpu.async_copy(
                    table_hbm.at[:, pl.ds(pl.multiple_of(c * 128, 128), 512)],
                    buf, sem)

        fire_group(0, buf0, sem0)
        fire_group(1, buf1, sem1)

        def init_sl(i, _):
            sl_v[pl.ds(i * 16, 16)] = jnp.full((16,), _B, jnp.int32)
            return 0
        lax.fori_loop(0, 8, init_sl, 0)

        def init_fill(i, _):
            fill_v[pl.ds(i * 16, 16)] = jnp.zeros((16,), jnp.int32)
            return 0
        lax.fori_loop(0, 16, init_fill, 0)

        pltpu.sync_copy(idx_hbm, idx_v)

        def bin_body(i, _):
            u = idx_v[pl.ds(i * 16, 16)]
            cc = lax.shift_right_logical(u, 7)
            mine = (cc >= c0) & (cc < cend)
            lb = cc - c0
            rank, last = plsc.scan_count(lb, mask=mine)
            base = plsc.load_gather(fill_v, [lb], mask=mine)
            slotb = base + rank - 1
            inb = mine & (slotb < 16)
            pack = (_iota16() + i * 16) | ((u & 127) << 14)
            plsc.store_scatter(bucket_v, [lb * 16 + slotb], pack, mask=inb)
            plsc.store_scatter(fill_v, [lb], base + rank, mask=mine & last)
            return 0

        lax.fori_loop(0, _B // 16, bin_body, 0)

        def ov_body(q, s):
            f16 = fill_v[pl.ds(q * 16, 16)]
            return s + jnp.sum(jnp.maximum(f16 - 16, 0))

        m_ov = lax.fori_loop(0, 16, ov_body, 0)

        def flush_reset(_sp):
            pltpu.async_copy(stage_v, out_hbm.at[sl_v], sem_s).wait()
            return 0

        def bucket_proc(lb, h, buf, sp):
            nh = plsc.load_gather(fill_v, [jnp.full((16,), 0, jnp.int32) + lb])
            sel = _iota16() < nh
            n = jnp.max(jnp.minimum(nh, 16))

            def do_group(sp):
                pp = bucket_v[pl.ds(lb * 16, 16)]
                bb = pp & 16383
                colv = lax.shift_right_logical(pp, 14) + h * 128
                slot = sp + plsc.cumsum(sel.astype(jnp.int32)) - 1
                for f in range(_D):
                    fvec = jnp.full((16,), f, jnp.int32)
                    vals = plsc.load_gather(buf, [fvec, colv], mask=sel)
                    plsc.store_scatter(stage_v, [slot, fvec], vals, mask=sel)
                plsc.store_scatter(sl_v, [slot], bb, mask=sel)
                sp = sp + n
                return lax.cond(sp >= 112, flush_reset, lambda s: s, sp)

            return lax.cond(n > 0, do_group, lambda s: s, sp)

        def process_group(g, buf, sp):
            sp = bucket_proc(4 * g, 0, buf, sp)
            for h in range(1, 4):
                lbh = 4 * g + h
                sp = lax.cond(
                    lbh < cn,
                    functools.partial(
                        lambda lbh, h, s: bucket_proc(lbh, h, buf, s),
                        lbh, h),
                    lambda s: s, sp)
            return sp

        def make_step(buf, sem):
            def step(g, sp):
                pltpu.make_async_copy(
                    table_hbm.at[:, pl.ds(0, 512)], buf, sem).wait()
                sp = process_group(g, buf, sp)

                @pl.when(g + 2 < ngroups)
                def _():
                    fire_group(g + 2, buf, sem)
                return sp
            return step

        step0 = make_step(buf0, sem0)
        step1 = make_step(buf1, sem1)

        def pair_body(p, sp):
            sp = lax.cond(2 * p < ngroups,
                          lambda s: step0(2 * p, s), lambda s: s, sp)
            sp = lax.cond(2 * p + 1 < ngroups,
                          lambda s: step1(2 * p + 1, s), lambda s: s, sp)
            return sp

        sp = lax.fori_loop(0, _GP, pair_body, 0)

        def fallback(sp):
            def slab_body(jj, sp):
                c = c0 + jj

                @pl.when(c == _TAIL_C)
                def _():
                    pltpu.sync_copy(tail_hbm, buf0.at[:, pl.ds(0, 128)])

                @pl.when(c != _TAIL_C)
                def _():
                    pltpu.sync_copy(
                        table_hbm.at[
                            :, pl.ds(pl.multiple_of(c * 128, 128), 128)],
                        buf0.at[:, pl.ds(0, 128)])

                def qbody(q, sp):
                    uu = idx_v[pl.ds(q * 16, 16)]
                    sel = lax.shift_right_logical(uu, 7) == c
                    n = jnp.max(plsc.all_reduce_population_count(sel))

                    def do_group(sp):
                        bb = _iota16() + q * 16
                        colv = uu & 127
                        slot = sp + plsc.cumsum(sel.astype(jnp.int32)) - 1
                        for f in range(_D):
                            fvec = jnp.full((16,), f, jnp.int32)
                            vals = plsc.load_gather(
                                buf0, [fvec, colv], mask=sel)
                            plsc.store_scatter(
                                stage_v, [slot, fvec], vals, mask=sel)
                        plsc.store_scatter(sl_v, [slot], bb, mask=sel)
                        sp = sp + n
                        return lax.cond(sp >= 112, flush_reset,
                                        lambda s: s, sp)

                    return lax.cond(n > 0, do_group, lambda s: s, sp)

                return lax.fori_loop(0, _B // 16, qbody, sp)

            return lax.fori_loop(0, cn, slab_body, sp)

        sp = lax.cond(m_ov > 0, fallback, lambda s: s, sp)

        pltpu.async_copy(stage_v, out_hbm.at[sl_v], sem_s).wait()

    return k(idx, table_t, tail)


def kernel(user_indices, visual_tokens):
    B = user_indices.shape[0]
    V, T, D = visual_tokens.shape
    table_t = visual_tokens.reshape(V * T, D).T
    tail = jnp.pad(table_t[:, _TAIL_U:], ((0, 0), (0, 128 - (V - _TAIL_U))))
    idx = user_indices.astype(jnp.int32)
    out3 = _gather_scan(idx, table_t, tail)
    return out3[:B, :D].reshape(B, T, D)
